# Initial kernel scaffold; baseline (speedup 1.0000x reference)
import jax
import jax.numpy as jnp
from jax.experimental import pallas as pl


def kernel(x, W1, b1, W2, b2, W3, b3):
    raise NotImplementedError("write your pallas kernel here")


# speedup vs baseline: 1.6571x; 1.6571x over previous
import numpy as np
from itertools import product

import jax
import jax.numpy as jnp
from jax.experimental import pallas as pl
from jax.experimental.pallas import tpu as pltpu

_IMAGE_INPUT_SIZE = 1024
_FMS = 32
_SCALES = [0.5, 1.0, 2.0]
_ASPECT_RATIOS = [0.5, 1.0, 2.0]
_K = 9
_HW = _FMS * _FMS
_NBOX = _HW * _K


def _anchor_boxes_np():
    d = _IMAGE_INPUT_SIZE / _FMS
    centers = np.arange(_FMS, dtype=np.float64) * d + d / 2.0
    cxg, cyg = np.meshgrid(centers, centers, indexing='ij')
    anchors = []
    for scale, ar in product(_SCALES, _ASPECT_RATIOS):
        w_new = float(int((ar ** 0.5) * scale * d))
        h_new = float(int(scale * d / (ar ** 0.5)))
        a = np.stack([cxg, cyg, np.full_like(cxg, w_new), np.full_like(cxg, h_new)], axis=-1)
        anchors.append(a[:, :, None, :])
    anchors = np.concatenate(anchors, axis=2)
    cx, cy, w, h = anchors[..., 0], anchors[..., 1], anchors[..., 2], anchors[..., 3]
    x1 = np.clip(cx - w / 2.0, 0.0, _IMAGE_INPUT_SIZE)
    y1 = np.clip(cy - h / 2.0, 0.0, _IMAGE_INPUT_SIZE)
    x2 = np.clip(cx + w / 2.0, 0.0, _IMAGE_INPUT_SIZE)
    y2 = np.clip(cy + h / 2.0, 0.0, _IMAGE_INPUT_SIZE)
    xywh = np.stack([x1, y1, x2 - x1, y2 - y1], axis=-1)
    return xywh.reshape(-1, 4).astype(np.float32)


def _decode_consts_np():
    a = _anchor_boxes_np().reshape(_HW, _K, 4).transpose(1, 2, 0)
    xa, ya, wa, ha = a[:, 0, :], a[:, 1, :], a[:, 2, :], a[:, 3, :]
    cxa = xa + wa / 2.0
    cya = ya + ha / 2.0
    z = np.zeros_like(wa)
    sc = np.stack([wa, ha, z, z], axis=1).reshape(36, _HW)
    bcst = np.stack([cxa, cya, z, z], axis=1).reshape(36, _HW)
    mc = np.stack([-wa / 2.0, -ha / 2.0, wa, ha], axis=1).reshape(36, _HW)
    perm = np.zeros((36, 36), np.float32)
    for l in range(36):
        perm[l, l + 2 if l % 4 < 2 else l] = 1.0
    return (sc.astype(np.float32), bcst.astype(np.float32),
            mc.astype(np.float32), perm)


def _fold_body(w1a_ref, w23_ref, b1_ref, b23_ref, wall_ref, bc_ref):
    w23 = w23_ref[...]
    rows = []
    for off in range(9):
        rows.append(jnp.dot(w23, w1a_ref[off * 256:(off + 1) * 256, :],
                            preferred_element_type=jnp.float32))
    wall_ref[...] = jnp.concatenate(rows, axis=0)
    bc = jnp.sum(w23 * b1_ref[...], axis=1, keepdims=True) + b23_ref[...]
    bc_ref[...] = jnp.broadcast_to(bc, (54, 128))


def _main_body(x_ref, wall_ref, bc_ref, sc_ref, bcst_ref, mc_ref, perm_ref,
               prop_ref, reg_ref, obj_ref):
    X = x_ref[0]
    y = jnp.dot(wall_ref[...], X, preferred_element_type=jnp.float32)
    p = jax.lax.broadcasted_iota(jnp.int32, (1, _HW), 1)
    h = p // _FMS
    w = p % _FMS
    acc = None
    for ky in range(3):
        for kx in range(3):
            off = ky * 3 + kx
            dy, dx = ky - 1, kx - 1
            s = dy * _FMS + dx
            yo = y[off * 54:(off + 1) * 54, :]
            if s != 0:
                yo = pltpu.roll(yo, (-s) % _HW, 1)
            mask = ((h + dy >= 0) & (h + dy < _FMS)
                    & (w + dx >= 0) & (w + dx < _FMS))
            contrib = jnp.where(mask, yo, 0.0)
            acc = contrib if acc is None else acc + contrib
    val = acc + bc_ref[:, 0:1]
    reg = val[:36, :]
    obj = val[36:54, :]
    c = jax.lax.broadcasted_iota(jnp.int32, (36, 1), 0) % 4
    e = jnp.exp(jnp.where(c < 2, 0.0, reg))
    esel = jnp.dot(perm_ref[...], e, preferred_element_type=jnp.float32)
    prop_ref[0] = reg * sc_ref[...] + bcst_ref[...] + mc_ref[...] * esel
    reg_ref[0] = reg
    obj_ref[0] = obj


def kernel(x, W1, b1, W2, b2, W3, b3):
    bsz = x.shape[0]
    f32 = jnp.float32
    x3 = x.reshape(bsz, 1024, _HW)
    w1a = jnp.transpose(W1, (2, 3, 0, 1)).reshape(9 * 256, 1024)
    w23 = jnp.concatenate([W2.reshape(36, 256), W3.reshape(18, 256)], axis=0)
    b1r = b1.reshape(1, 256)
    b23 = jnp.concatenate([b2, b3]).reshape(54, 1)

    wall, bc = pl.pallas_call(
        _fold_body,
        out_shape=[jax.ShapeDtypeStruct((486, 1024), f32),
                   jax.ShapeDtypeStruct((54, 128), f32)],
    )(w1a, w23, b1r, b23)

    sc_np, bcst_np, mc_np, perm_np = _decode_consts_np()
    sc = jnp.asarray(sc_np)
    bcst = jnp.asarray(bcst_np)
    mc = jnp.asarray(mc_np)
    perm = jnp.asarray(perm_np)

    prop36, reg36, obj18 = pl.pallas_call(
        _main_body,
        grid=(bsz,),
        in_specs=[
            pl.BlockSpec((1, 1024, _HW), lambda b: (b, 0, 0)),
            pl.BlockSpec((486, 1024), lambda b: (0, 0)),
            pl.BlockSpec((54, 128), lambda b: (0, 0)),
            pl.BlockSpec((36, _HW), lambda b: (0, 0)),
            pl.BlockSpec((36, _HW), lambda b: (0, 0)),
            pl.BlockSpec((36, _HW), lambda b: (0, 0)),
            pl.BlockSpec((36, 36), lambda b: (0, 0)),
        ],
        out_specs=[
            pl.BlockSpec((1, 36, _HW), lambda b: (b, 0, 0)),
            pl.BlockSpec((1, 36, _HW), lambda b: (b, 0, 0)),
            pl.BlockSpec((1, 18, _HW), lambda b: (b, 0, 0)),
        ],
        out_shape=[jax.ShapeDtypeStruct((bsz, 36, _HW), f32),
                   jax.ShapeDtypeStruct((bsz, 36, _HW), f32),
                   jax.ShapeDtypeStruct((bsz, 18, _HW), f32)],
    )(x3, wall, bc, sc, bcst, mc, perm)

    proposed = prop36.transpose(0, 2, 1).reshape(bsz, _NBOX, 4)
    reg_out = reg36.transpose(0, 2, 1).reshape(bsz, _NBOX, 4)
    obj_out = obj18.transpose(0, 2, 1).reshape(bsz, _NBOX, 2)
    anchors_out = jnp.tile(jnp.asarray(_anchor_boxes_np())[None, :, :], (bsz, 1, 1))
    return (proposed, reg_out, obj_out, anchors_out)
